# baseline (speedup 1.0000x reference)
import functools

import jax
import jax.numpy as jnp
from jax import lax
from jax.experimental import pallas as pl
from jax.experimental.pallas import tpu as pltpu
from jax.experimental.pallas import tpu_sc as plsc

B = 16384
D = 100
K = 100000
N = B * D
NC = 2
NS = 16
NW = NC * NS
PER_W = N // NW
CHUNK = 3200
QCH = 3 * CHUNK
N_CHUNKS = PER_W // CHUNK
LANES = 16

_mesh = plsc.VectorSubcoreMesh(core_axis_name="c", subcore_axis_name="s")


def _replicate3(vec, pats, comps3):
    dn = lax.GatherDimensionNumbers(
        offset_dims=(), collapsed_slice_dims=(0,), start_index_map=(0,)
    )
    outs = []
    for t in range(3):
        g = lax.gather(
            vec,
            pats[t][:, None],
            dn,
            slice_sizes=(1,),
            mode=lax.GatherScatterMode.PROMISE_IN_BOUNDS,
        )
        outs.append(g + comps3[t])
    return outs


@functools.partial(
    pl.kernel,
    mesh=_mesh,
    compiler_params=pltpu.CompilerParams(use_tc_tiling_on_sc=False),
    out_type=jax.ShapeDtypeStruct((N * 3,), jnp.float32),
    scratch_types=[
        pltpu.VMEM((CHUNK,), jnp.int32),
        pltpu.VMEM((QCH,), jnp.int32),
        pltpu.VMEM((QCH,), jnp.float32),
        pltpu.SemaphoreType.DMA,
    ],
)
def _sc_gather(labels_hbm, table_hbm, out_hbm, lab_v, idx3_v, val_v, sem):
    wid = lax.axis_index("s") * NC + lax.axis_index("c")
    base = wid * PER_W

    iota = lax.iota(jnp.int32, LANES)
    pats, comps3 = [], []
    for t in range(3):
        n = iota + LANES * t
        p = (n * 171) >> 9
        pats.append(p)
        comps3.append(n - 3 * p)

    def chunk_body(c, _):
        off = pl.multiple_of(base + c * CHUNK, CHUNK)
        pltpu.sync_copy(labels_hbm.at[pl.ds(off, CHUNK)], lab_v)

        def expand_body(m, _):
            s = pl.ds(pl.multiple_of(m * LANES, LANES), LANES)
            lab = lab_v[s]
            d = (iota + m * LANES) % D
            fidx3 = lab * 3 + d * (3 * K)
            outs = _replicate3(fidx3, pats, comps3)
            for t in range(3):
                s_out = pl.ds(
                    pl.multiple_of(m * (3 * LANES) + t * LANES, LANES), LANES
                )
                idx3_v[s_out] = outs[t]
            return 0

        lax.fori_loop(0, CHUNK // LANES, expand_body, 0)
        pltpu.async_copy(table_hbm.at[idx3_v], val_v, sem).wait()
        pltpu.sync_copy(val_v, out_hbm.at[pl.ds(off * 3, QCH)])
        return 0

    lax.fori_loop(0, N_CHUNKS, chunk_body, 0)


def kernel(labels, pqc_params):
    labels_flat = labels.astype(jnp.int32).reshape(N)
    table = pqc_params.reshape(D * K * 3)
    out = _sc_gather(labels_flat, table)
    return out.reshape(B, D, 3)


# speedup vs baseline: 1.0002x; 1.0002x over previous
import functools

import jax
import jax.numpy as jnp
from jax import lax
from jax.experimental import pallas as pl
from jax.experimental.pallas import tpu as pltpu
from jax.experimental.pallas import tpu_sc as plsc

B = 16384
D = 100
K = 100000
N = B * D
NC = 2
NS = 16
NW = NC * NS
PER_W = N // NW
CHUNK = 3200
QCH = 3 * CHUNK
N_CHUNKS = PER_W // CHUNK
LANES = 16

_mesh = plsc.VectorSubcoreMesh(core_axis_name="c", subcore_axis_name="s")


def _replicate3(vec, pats, comps3):
    dn = lax.GatherDimensionNumbers(
        offset_dims=(), collapsed_slice_dims=(0,), start_index_map=(0,)
    )
    outs = []
    for t in range(3):
        g = lax.gather(
            vec,
            pats[t][:, None],
            dn,
            slice_sizes=(1,),
            mode=lax.GatherScatterMode.PROMISE_IN_BOUNDS,
        )
        outs.append(g + comps3[t])
    return outs


@functools.partial(
    pl.kernel,
    mesh=_mesh,
    out_type=jax.ShapeDtypeStruct((N * 3,), jnp.float32),
    scratch_types=[
        pltpu.VMEM((CHUNK,), jnp.int32),
        pltpu.VMEM((QCH,), jnp.int32),
        pltpu.VMEM((QCH,), jnp.float32),
        pltpu.SemaphoreType.DMA,
    ],
)
def _sc_gather(labels_hbm, table_hbm, out_hbm, lab_v, idx3_v, val_v, sem):
    wid = lax.axis_index("s") * NC + lax.axis_index("c")
    base = wid * PER_W

    iota = lax.iota(jnp.int32, LANES)
    pats, comps3 = [], []
    for t in range(3):
        n = iota + LANES * t
        p = (n * 171) >> 9
        pats.append(p)
        comps3.append(n - 3 * p)

    def chunk_body(c, _):
        off = pl.multiple_of(base + c * CHUNK, CHUNK)
        pltpu.sync_copy(labels_hbm.at[pl.ds(off, CHUNK)], lab_v)

        def expand_body(m, _):
            s = pl.ds(pl.multiple_of(m * LANES, LANES), LANES)
            lab = lab_v[s]
            d = (iota + m * LANES) % D
            fidx3 = lab * 3 + d * (3 * K)
            outs = _replicate3(fidx3, pats, comps3)
            for t in range(3):
                s_out = pl.ds(
                    pl.multiple_of(m * (3 * LANES) + t * LANES, LANES), LANES
                )
                idx3_v[s_out] = outs[t]
            return 0

        lax.fori_loop(0, CHUNK // LANES, expand_body, 0)
        pltpu.async_copy(table_hbm.at[idx3_v], val_v, sem).wait()
        pltpu.sync_copy(val_v, out_hbm.at[pl.ds(off * 3, QCH)])
        return 0

    lax.fori_loop(0, N_CHUNKS, chunk_body, 0)


def kernel(labels, pqc_params):
    labels_flat = labels.astype(jnp.int32).reshape(N)
    table = pqc_params.reshape(D * K * 3)
    out = _sc_gather(labels_flat, table)
    return out.reshape(B, D, 3)


# speedup vs baseline: 20.4752x; 20.4715x over previous
import functools

import jax
import jax.numpy as jnp
from jax import lax
from jax.experimental import pallas as pl
from jax.experimental.pallas import tpu as pltpu
from jax.experimental.pallas import tpu_sc as plsc

B = 16384
D = 100
K = 100000
DK = D * K
DB = D * B
NC = 2
NS = 16
NW = NC * NS
PER_W = DB // NW
CHUNK = 6400
N_CHUNKS = PER_W // CHUNK
LANES = 16

_mesh = plsc.VectorSubcoreMesh(core_axis_name="c", subcore_axis_name="s")


@functools.partial(
    pl.kernel,
    mesh=_mesh,
    out_type=jax.ShapeDtypeStruct((3 * DB,), jnp.float32),
    scratch_types=[
        pltpu.VMEM((CHUNK,), jnp.int32),
        pltpu.VMEM((CHUNK,), jnp.int32),
        pltpu.VMEM((CHUNK,), jnp.int32),
        pltpu.VMEM((CHUNK,), jnp.int32),
        pltpu.VMEM((CHUNK,), jnp.float32),
        pltpu.VMEM((CHUNK,), jnp.float32),
        pltpu.VMEM((CHUNK,), jnp.float32),
        pltpu.SemaphoreType.DMA,
    ],
)
def _sc_gather(labt_hbm, table_hbm, out_hbm, lab_v, i0_v, i1_v, i2_v,
               v0_v, v1_v, v2_v, sem):
    wid = lax.axis_index("s") * NC + lax.axis_index("c")
    base = wid * PER_W

    iota = lax.iota(jnp.int32, LANES)

    def chunk_body(ch, _):
        off = pl.multiple_of(base + ch * CHUNK, CHUNK)
        pltpu.sync_copy(labt_hbm.at[pl.ds(off, CHUNK)], lab_v)

        def idx_body(m, _):
            s = pl.ds(pl.multiple_of(m * LANES, LANES), LANES)
            pos = iota + (off + m * LANES)
            d = pos >> 14
            i0 = lab_v[s] + d * K
            i0_v[s] = i0
            i1_v[s] = i0 + DK
            i2_v[s] = i0 + 2 * DK
            return 0

        lax.fori_loop(0, CHUNK // LANES, idx_body, 0)
        pltpu.async_copy(table_hbm.at[i0_v], v0_v, sem).wait()
        pltpu.async_copy(table_hbm.at[i1_v], v1_v, sem).wait()
        pltpu.async_copy(table_hbm.at[i2_v], v2_v, sem).wait()
        pltpu.sync_copy(v0_v, out_hbm.at[pl.ds(off, CHUNK)])
        pltpu.sync_copy(v1_v, out_hbm.at[pl.ds(DB + off, CHUNK)])
        pltpu.sync_copy(v2_v, out_hbm.at[pl.ds(2 * DB + off, CHUNK)])
        return 0

    lax.fori_loop(0, N_CHUNKS, chunk_body, 0)


def kernel(labels, pqc_params):
    labt = jnp.transpose(labels.astype(jnp.int32)).reshape(DB)
    table = jnp.transpose(pqc_params, (2, 0, 1)).reshape(3 * DK)
    out = _sc_gather(labt, table)
    return jnp.transpose(out.reshape(3, D, B), (2, 1, 0))


# speedup vs baseline: 97.3015x; 4.7522x over previous
import functools

import jax
import jax.numpy as jnp
from jax import lax
from jax.experimental import pallas as pl
from jax.experimental.pallas import tpu as pltpu
from jax.experimental.pallas import tpu_sc as plsc

B = 16384
D = 100
K = 100000
DB = D * B
NC = 2
NS = 16
NW = NC * NS
PER_W = DB // NW
CHUNK = 6400
N_CHUNKS = PER_W // CHUNK
LANES = 16

DPAD = 104
KPAD = 100096
DBLK = DPAD // 8
KBLK = KPAD // 128
PLANE_STRIDE = DBLK * KBLK * 1024
DBLK_STRIDE = KBLK * 1024

_mesh = plsc.VectorSubcoreMesh(core_axis_name="c", subcore_axis_name="s")


@functools.partial(
    pl.kernel,
    mesh=_mesh,
    out_type=jax.ShapeDtypeStruct((3 * DB,), jnp.float32),
    scratch_types=[
        pltpu.VMEM((CHUNK,), jnp.int32),
        pltpu.VMEM((CHUNK,), jnp.int32),
        pltpu.VMEM((CHUNK,), jnp.int32),
        pltpu.VMEM((CHUNK,), jnp.int32),
        pltpu.VMEM((CHUNK,), jnp.float32),
        pltpu.VMEM((CHUNK,), jnp.float32),
        pltpu.VMEM((CHUNK,), jnp.float32),
        pltpu.SemaphoreType.DMA,
    ],
)
def _sc_gather(labt_hbm, table_hbm, out_hbm, lab_v, i0_v, i1_v, i2_v,
               v0_v, v1_v, v2_v, sem):
    wid = lax.axis_index("s") * NC + lax.axis_index("c")
    base = wid * PER_W

    iota = lax.iota(jnp.int32, LANES)

    def chunk_body(ch, _):
        off = pl.multiple_of(base + ch * CHUNK, CHUNK)
        pltpu.sync_copy(labt_hbm.at[pl.ds(off, CHUNK)], lab_v)

        def idx_body(m, _):
            s = pl.ds(pl.multiple_of(m * LANES, LANES), LANES)
            pos = iota + (off + m * LANES)
            d = pos >> 14
            k = lab_v[s]
            kpart = ((k >> 7) << 10) + (k & 127)
            dpart = (d >> 3) * DBLK_STRIDE + ((d & 7) << 7)
            i0 = kpart + dpart
            i0_v[s] = i0
            i1_v[s] = i0 + PLANE_STRIDE
            i2_v[s] = i0 + 2 * PLANE_STRIDE
            return 0

        lax.fori_loop(0, CHUNK // LANES, idx_body, 0)
        pltpu.async_copy(table_hbm.at[i0_v], v0_v, sem).wait()
        pltpu.async_copy(table_hbm.at[i1_v], v1_v, sem).wait()
        pltpu.async_copy(table_hbm.at[i2_v], v2_v, sem).wait()
        pltpu.sync_copy(v0_v, out_hbm.at[pl.ds(off, CHUNK)])
        pltpu.sync_copy(v1_v, out_hbm.at[pl.ds(DB + off, CHUNK)])
        pltpu.sync_copy(v2_v, out_hbm.at[pl.ds(2 * DB + off, CHUNK)])
        return 0

    lax.fori_loop(0, N_CHUNKS, chunk_body, 0)


def kernel(labels, pqc_params):
    labt = jnp.transpose(labels.astype(jnp.int32)).reshape(DB)
    planes = jnp.transpose(pqc_params, (2, 0, 1))
    padded = jnp.pad(planes, ((0, 0), (0, DPAD - D), (0, KPAD - K)))
    tiles = padded.reshape(3, DBLK, 8, KBLK, 128).transpose(0, 1, 3, 2, 4)
    table = tiles.reshape(3 * PLANE_STRIDE)
    out = _sc_gather(labt, table)
    return jnp.transpose(out.reshape(3, D, B), (2, 1, 0))


# speedup vs baseline: 105.5052x; 1.0843x over previous
import functools

import jax
import jax.numpy as jnp
from jax import lax
from jax.experimental import pallas as pl
from jax.experimental.pallas import tpu as pltpu
from jax.experimental.pallas import tpu_sc as plsc

B = 16384
D = 100
K = 100000
DB = D * B
NC = 2
NS = 16
NW = NC * NS
PER_W = DB // NW
CHUNK = 6400
N_CHUNKS = PER_W // CHUNK
LANES = 16

DPAD = 104
KPAD = 100096
DBLK = DPAD // 8
KBLK = KPAD // 128
PLANE_STRIDE = DBLK * KBLK * 1024
DBLK_STRIDE = KBLK * 1024

_mesh = plsc.VectorSubcoreMesh(core_axis_name="c", subcore_axis_name="s")


@functools.partial(
    pl.kernel,
    mesh=_mesh,
    out_type=jax.ShapeDtypeStruct((3 * DB,), jnp.float32),
    scratch_types=[
        pltpu.VMEM((CHUNK,), jnp.int32),
        pltpu.VMEM((CHUNK,), jnp.int32),
        pltpu.VMEM((CHUNK,), jnp.int32),
        pltpu.VMEM((CHUNK,), jnp.int32),
        pltpu.VMEM((CHUNK,), jnp.int32),
        pltpu.VMEM((CHUNK,), jnp.int32),
        pltpu.VMEM((CHUNK,), jnp.int32),
        pltpu.VMEM((CHUNK,), jnp.int32),
        pltpu.VMEM((CHUNK,), jnp.float32),
        pltpu.VMEM((CHUNK,), jnp.float32),
        pltpu.VMEM((CHUNK,), jnp.float32),
        pltpu.SemaphoreType.DMA,
    ],
)
def _sc_gather(labt_hbm, table_hbm, out_hbm, lab_a, lab_b, i0_a, i1_a, i2_a,
               i0_b, i1_b, i2_b, v0_v, v1_v, v2_v, sem):
    wid = lax.axis_index("s") * NC + lax.axis_index("c")
    base = wid * PER_W

    iota = lax.iota(jnp.int32, LANES)
    bufs = ((lab_a, i0_a, i1_a, i2_a), (lab_b, i0_b, i1_b, i2_b))

    def load_and_index(ch, p):
        lab_v, i0_v, i1_v, i2_v = bufs[p]
        off = pl.multiple_of(base + ch * CHUNK, CHUNK)
        pltpu.sync_copy(labt_hbm.at[pl.ds(off, CHUNK)], lab_v)

        def idx_body(m, _):
            s = pl.ds(pl.multiple_of(m * LANES, LANES), LANES)
            pos = iota + (off + m * LANES)
            d = pos >> 14
            k = lab_v[s]
            kpart = ((k >> 7) << 10) + (k & 127)
            dpart = (d >> 3) * DBLK_STRIDE + ((d & 7) << 7)
            i0 = kpart + dpart
            i0_v[s] = i0
            i1_v[s] = i0 + PLANE_STRIDE
            i2_v[s] = i0 + 2 * PLANE_STRIDE
            return 0

        lax.fori_loop(0, CHUNK // LANES, idx_body, 0)

    load_and_index(0, 0)
    for ch in range(N_CHUNKS):
        p = ch % 2
        _, i0_v, i1_v, i2_v = bufs[p]
        off = base + ch * CHUNK
        cp0 = pltpu.async_copy(table_hbm.at[i0_v], v0_v, sem)
        cp1 = pltpu.async_copy(table_hbm.at[i1_v], v1_v, sem)
        cp2 = pltpu.async_copy(table_hbm.at[i2_v], v2_v, sem)
        if ch + 1 < N_CHUNKS:
            load_and_index(ch + 1, 1 - p)
        cp0.wait()
        cp1.wait()
        cp2.wait()
        pltpu.sync_copy(v0_v, out_hbm.at[pl.ds(off, CHUNK)])
        pltpu.sync_copy(v1_v, out_hbm.at[pl.ds(DB + off, CHUNK)])
        pltpu.sync_copy(v2_v, out_hbm.at[pl.ds(2 * DB + off, CHUNK)])


def kernel(labels, pqc_params):
    labt = jnp.transpose(labels.astype(jnp.int32)).reshape(DB)
    planes = jnp.transpose(pqc_params, (2, 0, 1))
    padded = jnp.pad(planes, ((0, 0), (0, DPAD - D), (0, KPAD - K)))
    tiles = padded.reshape(3, DBLK, 8, KBLK, 128).transpose(0, 1, 3, 2, 4)
    table = tiles.reshape(3 * PLANE_STRIDE)
    out = _sc_gather(labt, table)
    return jnp.transpose(out.reshape(3, D, B), (2, 1, 0))


# speedup vs baseline: 107.1096x; 1.0152x over previous
import functools

import jax
import jax.numpy as jnp
from jax import lax
from jax.experimental import pallas as pl
from jax.experimental.pallas import tpu as pltpu
from jax.experimental.pallas import tpu_sc as plsc

B = 16384
D = 100
K = 100000
DB = D * B
NC = 2
NS = 16
NW = NC * NS
PER_W = DB // NW
CHUNK = 6400
N_CHUNKS = PER_W // CHUNK
LANES = 16

DPAD = 104
KPAD = 100096
DBLK = DPAD // 8
KBLK = KPAD // 128
PLANE_STRIDE = DBLK * KBLK * 1024
DBLK_STRIDE = KBLK * 1024

_mesh = plsc.VectorSubcoreMesh(core_axis_name="c", subcore_axis_name="s")


@functools.partial(
    pl.kernel,
    mesh=_mesh,
    out_type=jax.ShapeDtypeStruct((3 * DB,), jnp.float32),
    scratch_types=[
        pltpu.VMEM((CHUNK,), jnp.int32),
        pltpu.VMEM((CHUNK,), jnp.int32),
        pltpu.VMEM((CHUNK,), jnp.int32),
        pltpu.VMEM((CHUNK,), jnp.int32),
        pltpu.VMEM((CHUNK,), jnp.int32),
        pltpu.VMEM((CHUNK,), jnp.int32),
        pltpu.VMEM((CHUNK,), jnp.int32),
        pltpu.VMEM((CHUNK,), jnp.int32),
        pltpu.VMEM((CHUNK,), jnp.float32),
        pltpu.VMEM((CHUNK,), jnp.float32),
        pltpu.VMEM((CHUNK,), jnp.float32),
        pltpu.VMEM((CHUNK,), jnp.float32),
        pltpu.VMEM((CHUNK,), jnp.float32),
        pltpu.VMEM((CHUNK,), jnp.float32),
        pltpu.SemaphoreType.DMA,
        pltpu.SemaphoreType.DMA,
    ],
)
def _sc_gather(labt_hbm, table_hbm, out_hbm, lab_a, lab_b, i0_a, i1_a, i2_a,
               i0_b, i1_b, i2_b, v0_a, v1_a, v2_a, v0_b, v1_b, v2_b,
               sem_a, sem_b):
    wid = lax.axis_index("s") * NC + lax.axis_index("c")
    base = wid * PER_W

    iota = lax.iota(jnp.int32, LANES)
    bufs = ((lab_a, i0_a, i1_a, i2_a), (lab_b, i0_b, i1_b, i2_b))

    def load_and_index(ch, p):
        lab_v, i0_v, i1_v, i2_v = bufs[p]
        off = pl.multiple_of(base + ch * CHUNK, CHUNK)
        pltpu.sync_copy(labt_hbm.at[pl.ds(off, CHUNK)], lab_v)

        def idx_body(m, _):
            s = pl.ds(pl.multiple_of(m * LANES, LANES), LANES)
            pos = iota + (off + m * LANES)
            d = pos >> 14
            k = lab_v[s]
            kpart = ((k >> 7) << 10) + (k & 127)
            dpart = (d >> 3) * DBLK_STRIDE + ((d & 7) << 7)
            i0 = kpart + dpart
            i0_v[s] = i0
            i1_v[s] = i0 + PLANE_STRIDE
            i2_v[s] = i0 + 2 * PLANE_STRIDE
            return 0

        lax.fori_loop(0, CHUNK // LANES, idx_body, 0)

    vbufs = ((v0_a, v1_a, v2_a, sem_a), (v0_b, v1_b, v2_b, sem_b))

    def drain_and_store(cps, ch):
        v0_v, v1_v, v2_v, _ = vbufs[ch % 2]
        off = base + ch * CHUNK
        for cp in cps:
            cp.wait()
        pltpu.sync_copy(v0_v, out_hbm.at[pl.ds(off, CHUNK)])
        pltpu.sync_copy(v1_v, out_hbm.at[pl.ds(DB + off, CHUNK)])
        pltpu.sync_copy(v2_v, out_hbm.at[pl.ds(2 * DB + off, CHUNK)])

    load_and_index(0, 0)
    prev = None
    for ch in range(N_CHUNKS):
        p = ch % 2
        _, i0_v, i1_v, i2_v = bufs[p]
        v0_v, v1_v, v2_v, sem = vbufs[p]
        cps = (
            pltpu.async_copy(table_hbm.at[i0_v], v0_v, sem),
            pltpu.async_copy(table_hbm.at[i1_v], v1_v, sem),
            pltpu.async_copy(table_hbm.at[i2_v], v2_v, sem),
        )
        if prev is not None:
            drain_and_store(prev, ch - 1)
        if ch + 1 < N_CHUNKS:
            load_and_index(ch + 1, 1 - p)
        prev = cps
    drain_and_store(prev, N_CHUNKS - 1)


def kernel(labels, pqc_params):
    labt = jnp.transpose(labels.astype(jnp.int32)).reshape(DB)
    planes = jnp.transpose(pqc_params, (2, 0, 1))
    padded = jnp.pad(planes, ((0, 0), (0, DPAD - D), (0, KPAD - K)))
    tiles = padded.reshape(3, DBLK, 8, KBLK, 128).transpose(0, 1, 3, 2, 4)
    table = tiles.reshape(3 * PLANE_STRIDE)
    out = _sc_gather(labt, table)
    return jnp.transpose(out.reshape(3, D, B), (2, 1, 0))
